# speedup vs baseline: 6.4299x; 6.4299x over previous
import functools

import jax
import jax.numpy as jnp
import numpy as np
from jax import lax
from jax.experimental import pallas as pl
from jax.experimental.pallas import tpu as pltpu

_PLANES = [32, 64, 128, 256, 512]
_STRIDE = [1, 4, 4, 4, 4]
_NSAMPLE = [8, 16, 16, 16, 16]
_SHARE = 8


def _fps_body(px_ref, py_ref, pz_ref, out_ref, *, m, rows):
    px = px_ref[...]
    py = py_ref[...]
    pz = pz_ref[...]
    n = rows * 128
    iota = (lax.broadcasted_iota(jnp.int32, (rows, 128), 0) * 128
            + lax.broadcasted_iota(jnp.int32, (rows, 128), 1))
    sel0 = iota == 0
    lx0 = jnp.sum(jnp.where(sel0, px, 0.0))
    ly0 = jnp.sum(jnp.where(sel0, py, 0.0))
    lz0 = jnp.sum(jnp.where(sel0, pz, 0.0))
    out_ref[0] = 0
    dist0 = jnp.full((rows, 128), 1e10, jnp.float32)

    def body(i, st):
        dist, lx, ly, lz = st
        dx = px - lx
        dy = py - ly
        dz = pz - lz
        d = (dx * dx + dy * dy) + dz * dz
        dist = jnp.minimum(dist, d)
        mx = jnp.max(dist)
        idx = jnp.min(jnp.where(dist == mx, iota, n))
        sel = iota == idx
        nlx = jnp.sum(jnp.where(sel, px, 0.0))
        nly = jnp.sum(jnp.where(sel, py, 0.0))
        nlz = jnp.sum(jnp.where(sel, pz, 0.0))
        out_ref[i] = idx
        return (dist, nlx, nly, nlz)

    lax.fori_loop(1, m, body, (dist0, lx0, ly0, lz0))


def _fps(p, m):
    n = p.shape[0]
    rows = n // 128
    px = p[:, 0].reshape(rows, 128)
    py = p[:, 1].reshape(rows, 128)
    pz = p[:, 2].reshape(rows, 128)
    return pl.pallas_call(
        functools.partial(_fps_body, m=m, rows=rows),
        out_shape=jax.ShapeDtypeStruct((m,), jnp.int32),
        out_specs=pl.BlockSpec(memory_space=pltpu.SMEM),
    )(px, py, pz)


def _knn_body(q_ref, rt_ref, idx_ref, dst_ref, *, k, nr_valid):
    q = q_ref[...]
    rt = rt_ref[...]
    bq = q.shape[0]
    nrp = rt.shape[1]
    qn = jnp.sum(q * q, axis=1, keepdims=True)
    rn = jnp.sum(rt * rt, axis=0, keepdims=True)
    d2 = (qn + rn) - 2.0 * jnp.dot(q, rt, preferred_element_type=jnp.float32)
    colio = lax.broadcasted_iota(jnp.int32, (bq, nrp), 1)
    if nr_valid < nrp:
        d2 = jnp.where(colio >= nr_valid, jnp.float32(3e38), d2)
    oio = lax.broadcasted_iota(jnp.int32, (bq, 128), 1)
    acc_i = jnp.zeros((bq, 128), jnp.int32)
    acc_d = jnp.zeros((bq, 128), jnp.float32)
    for j in range(k):
        mn = jnp.min(d2, axis=1, keepdims=True)
        am = jnp.min(jnp.where(d2 <= mn, colio, nrp), axis=1, keepdims=True)
        acc_i = jnp.where(oio == j, am, acc_i)
        acc_d = jnp.where(oio == j, jnp.sqrt(jnp.maximum(mn, 0.0)), acc_d)
        d2 = jnp.where(colio == am, jnp.float32(3e38), d2)
    idx_ref[...] = acc_i
    dst_ref[...] = acc_d


def _knn(q, r, k):
    nq = q.shape[0]
    nr = r.shape[0]
    nrp = max(128, nr)
    qp = jnp.zeros((nq, 8), jnp.float32).at[:, :3].set(q)
    rtp = jnp.zeros((8, nrp), jnp.float32).at[:3, :nr].set(r.T)
    bq = min(128, nq)
    grid = nq // bq
    idx, dst = pl.pallas_call(
        functools.partial(_knn_body, k=k, nr_valid=nr),
        grid=(grid,),
        in_specs=[
            pl.BlockSpec((bq, 8), lambda i: (i, 0)),
            pl.BlockSpec((8, nrp), lambda i: (0, 0)),
        ],
        out_specs=[
            pl.BlockSpec((bq, 128), lambda i: (i, 0)),
            pl.BlockSpec((bq, 128), lambda i: (i, 0)),
        ],
        out_shape=[
            jax.ShapeDtypeStruct((nq, 128), jnp.int32),
            jax.ShapeDtypeStruct((nq, 128), jnp.float32),
        ],
    )(qp, rtp)
    return idx[:, :k], dst[:, :k]


def _linear(x, p):
    y = x @ p["W"].T
    if "b" in p:
        y = y + p["b"]
    return y


def _bnorm(x, p, eps=1e-5):
    ax = (0,) if x.ndim == 2 else (0, 1)
    m = jnp.mean(x, axis=ax, keepdims=True)
    v = jnp.var(x, axis=ax, keepdims=True)
    return (x - m) / jnp.sqrt(v + eps) * p["g"] + p["b"]


def _transformer(prm, p, x, ns):
    n = x.shape[0]
    c = prm["v"]["W"].shape[0]
    xq = _linear(x, prm["q"])
    xk = _linear(x, prm["k"])
    xv = _linear(x, prm["v"])
    idx, _ = _knn(p, p, ns)
    pr = p[idx] - p[:, None, :]
    xk_g = xk[idx]
    xv_g = xv[idx]
    pe = _linear(jax.nn.relu(_bnorm(_linear(pr, prm["p1"]), prm["pbn"])), prm["p2"])
    r_qk = xk_g - xq[:, None, :] + pe.reshape(n, ns, -1, c).sum(2)
    w = jax.nn.relu(_bnorm(r_qk, prm["wbn1"]))
    w = _linear(w, prm["w1"])
    w = jax.nn.relu(_bnorm(w, prm["wbn2"]))
    w = _linear(w, prm["w2"])
    w = jax.nn.softmax(w, axis=1)
    v = (xv_g + pe).reshape(n, ns, _SHARE, c // _SHARE)
    return jnp.einsum('ntsi,nti->nsi', v, w).reshape(n, c)


def _bottleneck(prm, p, x, ns):
    identity = x
    h = jax.nn.relu(_bnorm(_linear(x, prm["l1"]), prm["bn1"]))
    h = jax.nn.relu(_bnorm(_transformer(prm["tr"], p, h, ns), prm["bn2"]))
    h = _bnorm(_linear(h, prm["l3"]), prm["bn3"])
    return jax.nn.relu(h + identity)


def _transition_down(prm, p, x, stride, ns):
    if stride == 1:
        return p, jax.nn.relu(_bnorm(_linear(x, prm["lin"]), prm["bn"]))
    m = p.shape[0] // stride
    sidx = _fps(p, m)
    np_ = p[sidx]
    kidx, _ = _knn(np_, p, ns)
    grouped = jnp.concatenate([p[kidx] - np_[:, None, :], x[kidx]], axis=-1)
    y = jax.nn.relu(_bnorm(_linear(grouped, prm["lin"]), prm["bn"]))
    return np_, y.max(axis=1)


def _interpolation(p2, p1, feat2):
    idx, dist = _knn(p1, p2, 3)
    w = 1.0 / (dist + 1e-8)
    w = w / w.sum(axis=1, keepdims=True)
    return (feat2[idx] * w[..., None]).sum(axis=1)


def kernel(p, x, o, params):
    feats = []
    cp, cx = p, x
    for i in range(5):
        prm = params["enc"][i]
        cp, cx = _transition_down(prm["td"], cp, cx, _STRIDE[i], _NSAMPLE[i])
        cx = _bottleneck(prm["blk"], cp, cx, _NSAMPLE[i])
        feats.append((cp, cx))
    p5, x5 = feats[4]
    tu = params["dec"][0]["tu"]
    glob = jax.nn.relu(_linear(jnp.mean(x5, axis=0, keepdims=True), tu["l2"]))
    h = jnp.concatenate(
        [x5, jnp.broadcast_to(glob, (x5.shape[0], glob.shape[1]))], axis=1)
    h = jax.nn.relu(_bnorm(_linear(h, tu["l1"]), tu["l1bn"]))
    cur_p, cur_x = p5, _bottleneck(params["dec"][0]["blk"], p5, h, _NSAMPLE[4])
    for j, i in enumerate(range(4, 0, -1)):
        p_f, x_f = feats[i - 1]
        tu = params["dec"][j + 1]["tu"]
        a = jax.nn.relu(_bnorm(_linear(x_f, tu["l1"]), tu["l1bn"]))
        b = _interpolation(
            cur_p, p_f,
            jax.nn.relu(_bnorm(_linear(cur_x, tu["l2"]), tu["l2bn"])))
        h = _bottleneck(params["dec"][j + 1]["blk"], p_f, a + b, _NSAMPLE[i - 1])
        cur_p, cur_x = p_f, h
    c = params["cls"]
    return _linear(jax.nn.relu(_bnorm(_linear(cur_x, c["l1"]), c["bn"])), c["l2"])


# speedup vs baseline: 6.4385x; 1.0013x over previous
import functools

import jax
import jax.numpy as jnp
import numpy as np
from jax import lax
from jax.experimental import pallas as pl
from jax.experimental.pallas import tpu as pltpu

_PLANES = [32, 64, 128, 256, 512]
_STRIDE = [1, 4, 4, 4, 4]
_NSAMPLE = [8, 16, 16, 16, 16]
_SHARE = 8


def _fps_body(px_ref, py_ref, pz_ref, out_ref, *, m, rows):
    px = px_ref[...]
    py = py_ref[...]
    pz = pz_ref[...]
    n = rows * 128
    iota = (lax.broadcasted_iota(jnp.int32, (rows, 128), 0) * 128
            + lax.broadcasted_iota(jnp.int32, (rows, 128), 1))
    sel0 = iota == 0
    lx0 = jnp.sum(jnp.where(sel0, px, 0.0))
    ly0 = jnp.sum(jnp.where(sel0, py, 0.0))
    lz0 = jnp.sum(jnp.where(sel0, pz, 0.0))
    out_ref[0] = 0
    dist0 = jnp.full((rows, 128), 1e10, jnp.float32)

    def body(i, st):
        dist, lx, ly, lz = st
        dx = px - lx
        dy = py - ly
        dz = pz - lz
        d = (dx * dx + dy * dy) + dz * dz
        dist = jnp.minimum(dist, d)
        mx = jnp.max(dist)
        idx = jnp.min(jnp.where(dist == mx, iota, n))
        sel = iota == idx
        nlx = jnp.sum(jnp.where(sel, px, 0.0))
        nly = jnp.sum(jnp.where(sel, py, 0.0))
        nlz = jnp.sum(jnp.where(sel, pz, 0.0))
        out_ref[i] = idx
        return (dist, nlx, nly, nlz)

    lax.fori_loop(1, m, body, (dist0, lx0, ly0, lz0))


def _fps(p, m):
    n = p.shape[0]
    rows = n // 128
    px = p[:, 0].reshape(rows, 128)
    py = p[:, 1].reshape(rows, 128)
    pz = p[:, 2].reshape(rows, 128)
    return pl.pallas_call(
        functools.partial(_fps_body, m=m, rows=rows),
        out_shape=jax.ShapeDtypeStruct((m,), jnp.int32),
        out_specs=pl.BlockSpec(memory_space=pltpu.SMEM),
    )(px, py, pz)


def _knn_body(q_ref, qn_ref, rt_ref, idx_ref, dst_ref, *, k, nr_valid):
    q = q_ref[...]
    qnb = qn_ref[...]
    rt = rt_ref[...]
    bq = q.shape[0]
    nrp = rt.shape[1]
    lane = lax.broadcasted_iota(jnp.int32, (bq, 128), 1)
    qn = jnp.sum(jnp.where(lane == 0, qnb, 0.0), axis=1, keepdims=True)
    rn = jnp.sum(rt * rt, axis=0, keepdims=True)
    prod = jnp.dot(q.astype(jnp.bfloat16), rt.astype(jnp.bfloat16),
                   preferred_element_type=jnp.float32)
    d2 = (qn + rn) - 2.0 * prod
    colio = lax.broadcasted_iota(jnp.int32, (bq, nrp), 1)
    if nr_valid < nrp:
        d2 = jnp.where(colio >= nr_valid, jnp.float32(3e38), d2)
    oio = lax.broadcasted_iota(jnp.int32, (bq, 128), 1)
    acc_i = jnp.zeros((bq, 128), jnp.int32)
    acc_d = jnp.zeros((bq, 128), jnp.float32)
    for j in range(k):
        mn = jnp.min(d2, axis=1, keepdims=True)
        am = jnp.min(jnp.where(d2 <= mn, colio, nrp), axis=1, keepdims=True)
        acc_i = jnp.where(oio == j, am, acc_i)
        acc_d = jnp.where(oio == j, mn, acc_d)
        d2 = jnp.where(colio == am, jnp.float32(3e38), d2)
    idx_ref[...] = acc_i
    dst_ref[...] = acc_d


def _knn(q, r, k):
    nq = q.shape[0]
    nr = r.shape[0]
    nrp = max(128, nr)
    qp = jnp.zeros((nq, 8), jnp.float32).at[:, :3].set(q)
    qnb = jnp.broadcast_to(jnp.sum(q * q, 1)[:, None], (nq, 128))
    rtp = jnp.zeros((8, nrp), jnp.float32).at[:3, :nr].set(r.T)
    bq = min(128, nq)
    grid = nq // bq
    idx, dst = pl.pallas_call(
        functools.partial(_knn_body, k=k, nr_valid=nr),
        grid=(grid,),
        in_specs=[
            pl.BlockSpec((bq, 8), lambda i: (i, 0)),
            pl.BlockSpec((bq, 128), lambda i: (i, 0)),
            pl.BlockSpec((8, nrp), lambda i: (0, 0)),
        ],
        out_specs=[
            pl.BlockSpec((bq, 128), lambda i: (i, 0)),
            pl.BlockSpec((bq, 128), lambda i: (i, 0)),
        ],
        out_shape=[
            jax.ShapeDtypeStruct((nq, 128), jnp.int32),
            jax.ShapeDtypeStruct((nq, 128), jnp.float32),
        ],
    )(qp, qnb, rtp)
    return idx[:, :k], jnp.sqrt(jnp.maximum(dst[:, :k], 0.0))


def _fps_jax(p, m):
    n = p.shape[0]
    idxs0 = jnp.zeros((m,), jnp.int32)
    dist0 = jnp.full((n,), 1e10, jnp.float32)

    def body(i, st):
        idxs, dist = st
        last = p[idxs[i - 1]]
        d = jnp.sum((p - last) ** 2, -1)
        dist = jnp.minimum(dist, d)
        idxs = idxs.at[i].set(jnp.argmax(dist).astype(jnp.int32))
        return (idxs, dist)

    idxs, _ = lax.fori_loop(1, m, body, (idxs0, dist0))
    return idxs


def _knn_jax(q, r, k):
    d2 = jnp.sum(q * q, 1)[:, None] + jnp.sum(r * r, 1)[None, :] - 2.0 * (q @ r.T)
    neg, idx = lax.top_k(-d2, k)
    return idx, jnp.sqrt(jnp.maximum(-neg, 0.0))


_fps_impl = _fps
_knn_impl = _knn


def _linear(x, p):
    y = x @ p["W"].T
    if "b" in p:
        y = y + p["b"]
    return y


def _bnorm(x, p, eps=1e-5):
    ax = (0,) if x.ndim == 2 else (0, 1)
    m = jnp.mean(x, axis=ax, keepdims=True)
    v = jnp.var(x, axis=ax, keepdims=True)
    return (x - m) / jnp.sqrt(v + eps) * p["g"] + p["b"]


def _transformer(prm, p, x, ns):
    n = x.shape[0]
    c = prm["v"]["W"].shape[0]
    xq = _linear(x, prm["q"])
    xk = _linear(x, prm["k"])
    xv = _linear(x, prm["v"])
    idx, _ = _knn(p, p, ns)
    pr = p[idx] - p[:, None, :]
    xk_g = xk[idx]
    xv_g = xv[idx]
    pe = _linear(jax.nn.relu(_bnorm(_linear(pr, prm["p1"]), prm["pbn"])), prm["p2"])
    r_qk = xk_g - xq[:, None, :] + pe.reshape(n, ns, -1, c).sum(2)
    w = jax.nn.relu(_bnorm(r_qk, prm["wbn1"]))
    w = _linear(w, prm["w1"])
    w = jax.nn.relu(_bnorm(w, prm["wbn2"]))
    w = _linear(w, prm["w2"])
    w = jax.nn.softmax(w, axis=1)
    v = (xv_g + pe).reshape(n, ns, _SHARE, c // _SHARE)
    return jnp.einsum('ntsi,nti->nsi', v, w).reshape(n, c)


def _bottleneck(prm, p, x, ns):
    identity = x
    h = jax.nn.relu(_bnorm(_linear(x, prm["l1"]), prm["bn1"]))
    h = jax.nn.relu(_bnorm(_transformer(prm["tr"], p, h, ns), prm["bn2"]))
    h = _bnorm(_linear(h, prm["l3"]), prm["bn3"])
    return jax.nn.relu(h + identity)


def _transition_down(prm, p, x, stride, ns):
    if stride == 1:
        return p, jax.nn.relu(_bnorm(_linear(x, prm["lin"]), prm["bn"]))
    m = p.shape[0] // stride
    sidx = _fps(p, m)
    np_ = p[sidx]
    kidx, _ = _knn(np_, p, ns)
    grouped = jnp.concatenate([p[kidx] - np_[:, None, :], x[kidx]], axis=-1)
    y = jax.nn.relu(_bnorm(_linear(grouped, prm["lin"]), prm["bn"]))
    return np_, y.max(axis=1)


def _interpolation(p2, p1, feat2):
    idx, dist = _knn(p1, p2, 3)
    w = 1.0 / (dist + 1e-8)
    w = w / w.sum(axis=1, keepdims=True)
    return (feat2[idx] * w[..., None]).sum(axis=1)


def kernel(p, x, o, params):
    return _kernel_body(p, x, o, params)


def _kernel_body(p, x, o, params):
    feats = []
    cp, cx = p, x
    for i in range(5):
        prm = params["enc"][i]
        cp, cx = _transition_down(prm["td"], cp, cx, _STRIDE[i], _NSAMPLE[i])
        cx = _bottleneck(prm["blk"], cp, cx, _NSAMPLE[i])
        feats.append((cp, cx))
    p5, x5 = feats[4]
    tu = params["dec"][0]["tu"]
    glob = jax.nn.relu(_linear(jnp.mean(x5, axis=0, keepdims=True), tu["l2"]))
    h = jnp.concatenate(
        [x5, jnp.broadcast_to(glob, (x5.shape[0], glob.shape[1]))], axis=1)
    h = jax.nn.relu(_bnorm(_linear(h, tu["l1"]), tu["l1bn"]))
    cur_p, cur_x = p5, _bottleneck(params["dec"][0]["blk"], p5, h, _NSAMPLE[4])
    for j, i in enumerate(range(4, 0, -1)):
        p_f, x_f = feats[i - 1]
        tu = params["dec"][j + 1]["tu"]
        a = jax.nn.relu(_bnorm(_linear(x_f, tu["l1"]), tu["l1bn"]))
        b = _interpolation(
            cur_p, p_f,
            jax.nn.relu(_bnorm(_linear(cur_x, tu["l2"]), tu["l2bn"])))
        h = _bottleneck(params["dec"][j + 1]["blk"], p_f, a + b, _NSAMPLE[i - 1])
        cur_p, cur_x = p_f, h
    c = params["cls"]
    return _linear(jax.nn.relu(_bnorm(_linear(cur_x, c["l1"]), c["bn"])), c["l2"])
